# Initial kernel scaffold; baseline (speedup 1.0000x reference)
import jax
import jax.numpy as jnp
from jax.experimental import pallas as pl


def kernel(user_emb, item_emb, a_indices, a_values, s_indices, s_values, item_r, item_c, enhance_weight, item_degree, batch_user, batch_pos, batch_neg):
    raise NotImplementedError("write your pallas kernel here")


# speedup vs baseline: 3.2000x; 3.2000x over previous
import jax
import jax.numpy as jnp
from jax import lax
from jax.experimental import pallas as pl
from jax.experimental.pallas import tpu as pltpu
from jax.experimental.pallas import tpu_sc as plsc

NU = 50000
NI = 50000
NT = 100000
H = 64
L = 16
HS = H // L
NC = 2
NS = 16
NW = NC * NS
CHUNK = 25088
RPT = CHUNK // NS
NB_A = 4
NB_S = 2
NT_PAD = NB_A * CHUNK
NU_PAD = NB_S * CHUNK
FB = 512
EBLK = 128
BATCH = 4096
N_NEG = 10
CONV = 10.0
_INTERPRET = False

_f32 = jnp.float32
_i32 = jnp.int32
_SC_PARAMS = pltpu.CompilerParams(needs_layout_passes=False,
                                  use_tc_tiling_on_sc=False)


def _mesh():
    return plsc.VectorSubcoreMesh(core_axis_name="c", subcore_axis_name="s",
                                  num_cores=NC, num_subcores=NS)


def _scal(x):
    x = jnp.asarray(x)
    return jnp.max(x) if x.ndim else x


def _lanes():
    return lax.broadcasted_iota(_i32, (L,), 0)


def _make_prepass(nb, nblk_in, src_off):
    cap = (nblk_in + 1) * FB
    out_type = (
        jax.ShapeDtypeStruct((nb, NW, cap), _i32),
        jax.ShapeDtypeStruct((nb, NW, cap), _i32),
        jax.ShapeDtypeStruct((nb, NW, cap), _f32),
        jax.ShapeDtypeStruct((NW, L), _i32),
    )
    scratch = [
        pltpu.VMEM((FB,), _i32),
        pltpu.VMEM((FB,), _i32),
        pltpu.VMEM((FB,), _f32),
    ]
    scratch += [pltpu.VMEM((2 * FB,), _i32) for _ in range(nb)]
    scratch += [pltpu.VMEM((2 * FB,), _i32) for _ in range(nb)]
    scratch += [pltpu.VMEM((2 * FB,), _f32) for _ in range(nb)]
    scratch.append(pltpu.VMEM((L,), _i32))

    def body(dst_h, src_h, val_h, bsrc, bdst, bval, cnts, ind, ins, inv, *rest):
        stg_s = rest[0:nb]
        stg_d = rest[nb:2 * nb]
        stg_v = rest[2 * nb:3 * nb]
        cntv = rest[3 * nb]
        w = lax.axis_index("s") * NC + lax.axis_index("c")
        pos = _lanes()

        def flush(b, nf):
            pltpu.sync_copy(stg_s[b].at[pl.ds(0, FB)],
                            bsrc.at[b, w, pl.ds(nf * FB, FB)])
            pltpu.sync_copy(stg_d[b].at[pl.ds(0, FB)],
                            bdst.at[b, w, pl.ds(nf * FB, FB)])
            pltpu.sync_copy(stg_v[b].at[pl.ds(0, FB)],
                            bval.at[b, w, pl.ds(nf * FB, FB)])

        def blk(j, carry):
            ptrs = list(carry[:nb])
            nfs = list(carry[nb:])
            pltpu.sync_copy(dst_h.at[w, j], ind)
            pltpu.sync_copy(src_h.at[w, j], ins)
            pltpu.sync_copy(val_h.at[w, j], inv)
            for v in range(FB // L):
                sl = pl.ds(v * L, L)
                dv = ind[sl]
                sv = ins[sl] + src_off if src_off else ins[sl]
                vv = inv[sl]
                bv = lax.div(dv, jnp.int32(CHUNK))
                dl = dv - bv * CHUNK
                for b in range(nb):
                    m = bv == b
                    incl = plsc.cumsum(m.astype(_i32))
                    tgt = ptrs[b] + incl - 1
                    plsc.store_scatter(stg_s[b], [tgt], sv, mask=m)
                    plsc.store_scatter(stg_d[b], [tgt], dl, mask=m)
                    plsc.store_scatter(stg_v[b], [tgt], vv, mask=m)
                    ptrs[b] = ptrs[b] + jnp.max(incl)
            for b in range(nb):
                fl = ptrs[b] >= FB

                @pl.when(fl)
                def _(b=b, nf=nfs[b]):
                    flush(b, nf)
                    for v in range(FB // L):
                        lo = pl.ds(v * L, L)
                        hi = pl.ds(FB + v * L, L)
                        stg_s[b][lo] = stg_s[b][hi]
                        stg_d[b][lo] = stg_d[b][hi]
                        stg_v[b][lo] = stg_v[b][hi]

                ptrs[b] = jnp.where(fl, ptrs[b] - FB, ptrs[b])
                nfs[b] = jnp.where(fl, nfs[b] + 1, nfs[b])
            return tuple(ptrs) + tuple(nfs)

        init = tuple(jnp.int32(0) for _ in range(2 * nb))
        carry = lax.fori_loop(0, nblk_in, blk, init)
        ptrs = carry[:nb]
        nfs = carry[nb:]
        cv = jnp.zeros((L,), _i32)
        for b in range(nb):
            ptr = ptrs[b]
            for v in range(FB // L):
                sl = pl.ds(v * L, L)
                gpos = pos + v * L
                mi = (gpos >= ptr).astype(_i32)
                mf = mi.astype(_f32)
                stg_v[b][sl] = stg_v[b][sl] * (1.0 - mf)
                stg_d[b][sl] = stg_d[b][sl] * (1 - mi) + gpos * mi
                stg_s[b][sl] = stg_s[b][sl] * (1 - mi) + gpos * mi
            flush(b, nfs[b])
            cv = jnp.where(pos == b,
                           jnp.where(ptr > 0, nfs[b] + 1, nfs[b]), cv)
        cntv[...] = cv
        pltpu.sync_copy(cntv, cnts.at[w])

    return pl.kernel(body, out_type=out_type, mesh=_mesh(),
                     compiler_params=_SC_PARAMS,
                     scratch_types=scratch, interpret=_INTERPRET)


def _make_spmm(nb, mode):
    nbs = nb // NC
    out_type = jax.ShapeDtypeStruct((nb * CHUNK, H), _f32)
    scratch = [
        pltpu.VMEM_SHARED((CHUNK, H), _f32),
        pltpu.VMEM((EBLK,), _i32),
        pltpu.VMEM((EBLK,), _i32),
        pltpu.VMEM((EBLK,), _f32),
        pltpu.VMEM((EBLK, H), _f32),
        pltpu.VMEM((EBLK, H), _f32),
        pltpu.VMEM((L,), _i32),
        pltpu.SemaphoreType.DMA,
    ]
    if mode != "plain":
        scratch += [pltpu.VMEM((32, H), _f32), pltpu.VMEM((32, H), _f32)]
    if mode == "items":
        scratch.append(pltpu.VMEM((32,), _f32))

    def body(*args):
        x_h, bsrc, bdst, bval, cnts = args[:5]
        args = args[5:]
        if mode == "items":
            deg_h = args[0]
            args = args[1:]
        out = args[0]
        accum, esrc, edst, evalv, rows, zbuf, cntv, sem = args[1:9]
        if mode != "plain":
            abuf, ybuf = args[9:11]
        if mode == "items":
            dbuf = args[11]
        c = lax.axis_index("c")
        s = lax.axis_index("s")
        pos = _lanes()
        zero = jnp.zeros((L,), _f32)
        for r0 in range(EBLK):
            for h0 in range(HS):
                zbuf[r0, pl.ds(h0 * L, L)] = zero
        for k in range(nbs):
            b = c * nbs + k
            for t in range(RPT // EBLK):
                pltpu.sync_copy(zbuf,
                                accum.at[pl.ds(s * RPT + t * EBLK, EBLK)])
            rem = RPT % EBLK
            if rem:
                pltpu.sync_copy(
                    zbuf.at[pl.ds(0, rem)],
                    accum.at[pl.ds(s * RPT + (RPT // EBLK) * EBLK, rem)])
            plsc.subcore_barrier()
            for rr in range(NW // NS):
                r = s * (NW // NS) + rr
                pltpu.sync_copy(cnts.at[r], cntv)
                nblk = jnp.max(jnp.where(pos == b, cntv[...], 0)) * (FB // EBLK)

                def jbody(j, _, b=b, r=r):
                    pltpu.sync_copy(bsrc.at[b, r, pl.ds(j * EBLK, EBLK)], esrc)
                    pltpu.sync_copy(bdst.at[b, r, pl.ds(j * EBLK, EBLK)], edst)
                    pltpu.sync_copy(bval.at[b, r, pl.ds(j * EBLK, EBLK)], evalv)
                    pltpu.async_copy(x_h.at[esrc], rows, sem).wait()
                    for row in range(EBLK):
                        wv = plsc.load_gather(evalv,
                                              [jnp.full((L,), row, _i32)])
                        for h0 in range(HS):
                            sl = pl.ds(h0 * L, L)
                            rows[row, sl] = rows[row, sl] * wv
                    pltpu.sync_copy(rows, accum.at[edst], add=True)
                    return 0

                lax.fori_loop(0, nblk, jbody, 0)
            plsc.subcore_barrier()
            if mode == "plain":
                pltpu.sync_copy(accum.at[pl.ds(s * RPT, RPT)],
                                out.at[pl.ds(b * CHUNK + s * RPT, RPT)])
            else:
                yoff = NU if mode == "items" else 0

                def ep(t, _, b=b):
                    lo = s * RPT + t * 32
                    glob = b * CHUNK + lo
                    pltpu.sync_copy(accum.at[pl.ds(lo, 32)], abuf)
                    pltpu.sync_copy(x_h.at[pl.ds(yoff + glob, 32)], ybuf)
                    if mode == "items":
                        pltpu.sync_copy(deg_h.at[pl.ds(glob, 32)], dbuf)
                    for row in range(32):
                        if mode == "users":
                            for h0 in range(HS):
                                sl = pl.ds(h0 * L, L)
                                abuf[row, sl] = (abuf[row, sl]
                                                 + ybuf[row, sl]) * 0.5
                        else:
                            dv = plsc.load_gather(
                                dbuf, [jnp.full((L,), row, _i32)])
                            sw = CONV / (CONV + jnp.exp(dv * (1.0 / CONV)))
                            for h0 in range(HS):
                                sl = pl.ds(h0 * L, L)
                                abuf[row, sl] = (ybuf[row, sl]
                                                 + sw * abuf[row, sl])
                    pltpu.sync_copy(abuf, out.at[pl.ds(glob, 32)])
                    return 0

                lax.fori_loop(0, RPT // 32, ep, 0)
            plsc.subcore_barrier()

    return pl.kernel(body, out_type=out_type, mesh=_mesh(),
                     compiler_params=_SC_PARAMS,
                     scratch_types=scratch, interpret=_INTERPRET)


def _make_bpr():
    G = 16
    out_type = (
        jax.ShapeDtypeStruct((BATCH, L), _f32),
        jax.ShapeDtypeStruct((BATCH, L), _f32),
        jax.ShapeDtypeStruct((NW, L), _f32),
    )
    scratch = [
        pltpu.VMEM((G,), _i32),
        pltpu.VMEM((G,), _i32),
        pltpu.VMEM((G * N_NEG,), _i32),
        pltpu.VMEM((G, H), _f32),
        pltpu.VMEM((G, H), _f32),
        pltpu.VMEM((G, H), _f32),
        pltpu.VMEM((G, H), _f32),
        pltpu.VMEM((G * N_NEG, H), _f32),
        pltpu.VMEM((G * N_NEG, H), _f32),
        pltpu.VMEM((G, L), _f32),
        pltpu.VMEM((G, L), _f32),
        pltpu.VMEM((L,), _f32),
        pltpu.SemaphoreType.DMA,
    ]

    def body(uf, itf, ue_h, ie_h, bu, bp, bn, pos_o, neg_o, reg_o,
             biu, bip, binn, xuf, xue, xitp, xiep, xitn, xien, spos, sneg,
             regv, sem):
        w = lax.axis_index("s") * NC + lax.axis_index("c")
        per_w = BATCH // NW

        def g_body(g, racc):
            e0 = w * per_w + g * G
            pltpu.sync_copy(bu.at[pl.ds(e0, G)], biu)
            pltpu.sync_copy(bp.at[pl.ds(e0, G)], bip)
            pltpu.sync_copy(bn.at[pl.ds(e0 * N_NEG, G * N_NEG)], binn)
            pltpu.async_copy(uf.at[biu], xuf, sem).wait()
            pltpu.async_copy(ue_h.at[biu], xue, sem).wait()
            pltpu.async_copy(itf.at[bip], xitp, sem).wait()
            pltpu.async_copy(ie_h.at[bip], xiep, sem).wait()
            pltpu.async_copy(itf.at[binn], xitn, sem).wait()
            pltpu.async_copy(ie_h.at[binn], xien, sem).wait()
            for e in range(G):
                pacc = jnp.zeros((L,), _f32)
                nacc = jnp.zeros((L,), _f32)
                for h0 in range(HS):
                    sl = pl.ds(h0 * L, L)
                    uv = xuf[e, sl]
                    pacc = pacc + uv * xitp[e, sl]
                    ev = xue[e, sl]
                    pv = xiep[e, sl]
                    racc = racc + ev * ev + pv * pv
                    for jn in range(N_NEG):
                        nacc = nacc + uv * xitn[e * N_NEG + jn, sl]
                        nv = xien[e * N_NEG + jn, sl]
                        racc = racc + nv * nv
                spos[e, pl.ds(0, L)] = pacc
                sneg[e, pl.ds(0, L)] = nacc
            pltpu.sync_copy(spos, pos_o.at[pl.ds(e0, G)])
            pltpu.sync_copy(sneg, neg_o.at[pl.ds(e0, G)])
            return racc

        racc = lax.fori_loop(0, per_w // G, g_body, jnp.zeros((L,), _f32))
        regv[...] = racc
        pltpu.sync_copy(regv, reg_o.at[w])

    return pl.kernel(body, out_type=out_type, mesh=_mesh(),
                     compiler_params=_SC_PARAMS,
                     scratch_types=scratch, interpret=_INTERPRET)


def _tc_final(pos_ref, neg_ref, reg_ref, loss_ref, regl_ref):
    pos = jnp.sum(pos_ref[...], axis=1)
    neg = jnp.sum(neg_ref[...], axis=1) * (1.0 / N_NEG)
    d = neg - pos
    sp = jnp.maximum(d, 0.0) + jnp.log1p(jnp.exp(-jnp.abs(d)))
    loss_ref[...] = jnp.mean(sp).reshape(1, 1)
    regl_ref[...] = (0.5 * jnp.sum(reg_ref[...]) / BATCH).reshape(1, 1)


def _pad_edges(dst, src, val, nblk_in, spread_mod):
    et = nblk_in * FB
    n = dst.shape[0]
    padn = NW * et - n
    ar = jnp.arange(padn, dtype=_i32)
    sp = ar % spread_mod
    dst = jnp.concatenate([dst.astype(_i32), sp])
    src = jnp.concatenate([src.astype(_i32), sp])
    val = jnp.concatenate([val.astype(_f32), jnp.zeros((padn,), _f32)])
    return (dst.reshape(NW, nblk_in, FB), src.reshape(NW, nblk_in, FB),
            val.reshape(NW, nblk_in, FB))


def kernel(user_emb, item_emb, a_indices, a_values, s_indices, s_values,
           item_r, item_c, enhance_weight, item_degree,
           batch_user, batch_pos, batch_neg):
    x0 = jnp.concatenate(
        [user_emb.astype(_f32), item_emb.astype(_f32),
         jnp.zeros((NT_PAD - NT, H), _f32)], axis=0)
    nbi_a = -(-(1000000 // NW) // FB)
    nbi_s = -(-(500000 // NW) // FB)
    nbi_i = -(-(200000 // NW) // FB)

    ad, asx, av = _pad_edges(a_indices[0], a_indices[1], a_values, nbi_a,
                             CHUNK)
    bs_a, bd_a, bv_a, cn_a = _make_prepass(NB_A, nbi_a, 0)(ad, asx, av)
    spmm_a = _make_spmm(NB_A, "plain")
    y1 = spmm_a(x0, bs_a, bd_a, bv_a, cn_a)
    y2 = spmm_a(y1, bs_a, bd_a, bv_a, cn_a)

    sd, ssx, sv = _pad_edges(s_indices[0], s_indices[1], s_values, nbi_s,
                             CHUNK)
    bs_s, bd_s, bv_s, cn_s = _make_prepass(NB_S, nbi_s, 0)(sd, ssx, sv)
    uf = _make_spmm(NB_S, "users")(y2, bs_s, bd_s, bv_s, cn_s)

    idd, isx, iv = _pad_edges(item_r, item_c, enhance_weight, nbi_i, CHUNK)
    bs_i, bd_i, bv_i, cn_i = _make_prepass(NB_S, nbi_i, NU)(idd, isx, iv)
    degp = jnp.concatenate(
        [item_degree.astype(_f32), jnp.zeros((NU_PAD - NI,), _f32)])
    itf = _make_spmm(NB_S, "items")(y2, bs_i, bd_i, bv_i, cn_i, degp)

    pos_p, neg_p, reg_p = _make_bpr()(
        uf, itf, user_emb.astype(_f32), item_emb.astype(_f32),
        batch_user.astype(_i32), batch_pos.astype(_i32),
        batch_neg.reshape(-1).astype(_i32))
    outs = pl.pallas_call(
        _tc_final,
        out_shape=(jax.ShapeDtypeStruct((1, 1), _f32),
                   jax.ShapeDtypeStruct((1, 1), _f32)),
        interpret=_INTERPRET,
    )(pos_p, neg_p, reg_p)
    return (outs[0][0, 0], outs[1][0, 0])


# speedup vs baseline: 3.7819x; 1.1818x over previous
import jax
import jax.numpy as jnp
from jax import lax
from jax.experimental import pallas as pl
from jax.experimental.pallas import tpu as pltpu
from jax.experimental.pallas import tpu_sc as plsc

NU = 50000
NI = 50000
NT = 100000
H = 64
L = 16
HS = H // L
NC = 2
NS = 16
NW = NC * NS
CHUNK = 16896
RPT = CHUNK // NS
NB_A = 6
NB_S = 4
NT_PAD = NB_A * CHUNK
NU_PAD = NB_S * CHUNK
FB = 512
EBLK = 128
BATCH = 4096
N_NEG = 10
CONV = 10.0
_INTERPRET = False

_f32 = jnp.float32
_i32 = jnp.int32
_SC_PARAMS = pltpu.CompilerParams(needs_layout_passes=False,
                                  use_tc_tiling_on_sc=False)


def _mesh():
    return plsc.VectorSubcoreMesh(core_axis_name="c", subcore_axis_name="s",
                                  num_cores=NC, num_subcores=NS)


def _scal(x):
    x = jnp.asarray(x)
    return jnp.max(x) if x.ndim else x


def _lanes():
    return lax.broadcasted_iota(_i32, (L,), 0)


def _make_prepass(nb, nblk_in, src_off):
    cap = (nblk_in + 1) * FB
    out_type = (
        jax.ShapeDtypeStruct((nb, NW, cap), _i32),
        jax.ShapeDtypeStruct((nb, NW, cap), _i32),
        jax.ShapeDtypeStruct((nb, NW, cap), _f32),
        jax.ShapeDtypeStruct((NW, L), _i32),
    )
    scratch = [
        pltpu.VMEM((FB,), _i32),
        pltpu.VMEM((FB,), _i32),
        pltpu.VMEM((FB,), _f32),
    ]
    scratch += [pltpu.VMEM((2 * FB,), _i32) for _ in range(nb)]
    scratch += [pltpu.VMEM((2 * FB,), _i32) for _ in range(nb)]
    scratch += [pltpu.VMEM((2 * FB,), _f32) for _ in range(nb)]
    scratch.append(pltpu.VMEM((L,), _i32))

    def body(dst_h, src_h, val_h, bsrc, bdst, bval, cnts, ind, ins, inv, *rest):
        stg_s = rest[0:nb]
        stg_d = rest[nb:2 * nb]
        stg_v = rest[2 * nb:3 * nb]
        cntv = rest[3 * nb]
        w = lax.axis_index("s") * NC + lax.axis_index("c")
        pos = _lanes()

        def flush(b, nf):
            pltpu.sync_copy(stg_s[b].at[pl.ds(0, FB)],
                            bsrc.at[b, w, pl.ds(nf * FB, FB)])
            pltpu.sync_copy(stg_d[b].at[pl.ds(0, FB)],
                            bdst.at[b, w, pl.ds(nf * FB, FB)])
            pltpu.sync_copy(stg_v[b].at[pl.ds(0, FB)],
                            bval.at[b, w, pl.ds(nf * FB, FB)])

        def blk(j, carry):
            ptrs = list(carry[:nb])
            nfs = list(carry[nb:])
            pltpu.sync_copy(dst_h.at[w, j], ind)
            pltpu.sync_copy(src_h.at[w, j], ins)
            pltpu.sync_copy(val_h.at[w, j], inv)
            for v in range(FB // L):
                sl = pl.ds(v * L, L)
                dv = ind[sl]
                sv = ins[sl] + src_off if src_off else ins[sl]
                vv = inv[sl]
                bv = lax.div(dv, jnp.int32(CHUNK))
                dl = dv - bv * CHUNK
                for b in range(nb):
                    m = bv == b
                    incl = plsc.cumsum(m.astype(_i32))
                    tgt = ptrs[b] + incl - 1
                    plsc.store_scatter(stg_s[b], [tgt], sv, mask=m)
                    plsc.store_scatter(stg_d[b], [tgt], dl, mask=m)
                    plsc.store_scatter(stg_v[b], [tgt], vv, mask=m)
                    ptrs[b] = ptrs[b] + jnp.max(incl)
            for b in range(nb):
                fl = ptrs[b] >= FB

                @pl.when(fl)
                def _(b=b, nf=nfs[b]):
                    flush(b, nf)
                    for v in range(FB // L):
                        lo = pl.ds(v * L, L)
                        hi = pl.ds(FB + v * L, L)
                        stg_s[b][lo] = stg_s[b][hi]
                        stg_d[b][lo] = stg_d[b][hi]
                        stg_v[b][lo] = stg_v[b][hi]

                ptrs[b] = jnp.where(fl, ptrs[b] - FB, ptrs[b])
                nfs[b] = jnp.where(fl, nfs[b] + 1, nfs[b])
            return tuple(ptrs) + tuple(nfs)

        init = tuple(jnp.int32(0) for _ in range(2 * nb))
        carry = lax.fori_loop(0, nblk_in, blk, init)
        ptrs = carry[:nb]
        nfs = carry[nb:]
        cv = jnp.zeros((L,), _i32)
        for b in range(nb):
            ptr = ptrs[b]
            for v in range(FB // L):
                sl = pl.ds(v * L, L)
                gpos = pos + v * L
                mi = (gpos >= ptr).astype(_i32)
                mf = mi.astype(_f32)
                stg_v[b][sl] = stg_v[b][sl] * (1.0 - mf)
                stg_d[b][sl] = stg_d[b][sl] * (1 - mi) + gpos * mi
                stg_s[b][sl] = stg_s[b][sl] * (1 - mi) + gpos * mi
            flush(b, nfs[b])
            cv = jnp.where(pos == b,
                           jnp.where(ptr > 0, nfs[b] + 1, nfs[b]), cv)
        cntv[...] = cv
        pltpu.sync_copy(cntv, cnts.at[w])

    return pl.kernel(body, out_type=out_type, mesh=_mesh(),
                     compiler_params=_SC_PARAMS,
                     scratch_types=scratch, interpret=_INTERPRET)


def _make_spmm(nb, mode):
    nbs = nb // NC
    NBUF = 4
    out_type = jax.ShapeDtypeStruct((nb * CHUNK, H), _f32)
    scratch = [pltpu.VMEM_SHARED((CHUNK, H), _f32)]
    scratch += [pltpu.VMEM((EBLK,), _i32) for _ in range(NBUF)]
    scratch += [pltpu.VMEM((EBLK,), _i32) for _ in range(NBUF)]
    scratch += [pltpu.VMEM((EBLK,), _f32) for _ in range(NBUF)]
    scratch += [pltpu.VMEM((EBLK, H), _f32) for _ in range(NBUF)]
    scratch += [pltpu.VMEM((32, H), _f32), pltpu.VMEM((L,), _i32)]
    scratch += [pltpu.SemaphoreType.DMA for _ in range(3 * NBUF)]
    if mode != "plain":
        scratch += [pltpu.VMEM((32, H), _f32), pltpu.VMEM((32, H), _f32)]
    if mode == "items":
        scratch.append(pltpu.VMEM((32,), _f32))

    def body(*args):
        x_h, bsrc, bdst, bval, cnts = args[:5]
        args = args[5:]
        if mode == "items":
            deg_h = args[0]
            args = args[1:]
        out = args[0]
        args = args[1:]
        accum = args[0]
        esrcs = args[1:1 + NBUF]
        edsts = args[1 + NBUF:1 + 2 * NBUF]
        evals = args[1 + 2 * NBUF:1 + 3 * NBUF]
        rowss = args[1 + 3 * NBUF:1 + 4 * NBUF]
        zbuf = args[1 + 4 * NBUF]
        cntv = args[2 + 4 * NBUF]
        base = 3 + 4 * NBUF
        sem_e = args[base:base + NBUF]
        sem_g = args[base + NBUF:base + 2 * NBUF]
        sem_s = args[base + 2 * NBUF:base + 3 * NBUF]
        rest = args[base + 3 * NBUF:]
        if mode != "plain":
            abuf, ybuf = rest[0], rest[1]
        if mode == "items":
            dbuf = rest[2]
        c = lax.axis_index("c")
        s = lax.axis_index("s")
        pos = _lanes()
        zero = jnp.zeros((L,), _f32)
        for r0 in range(32):
            for h0 in range(HS):
                zbuf[r0, pl.ds(h0 * L, L)] = zero
        for k in range(nbs):
            b = c * nbs + k

            def zr(t, _):
                pltpu.sync_copy(zbuf, accum.at[pl.ds(s * RPT + t * 32, 32)])
                return 0

            lax.fori_loop(0, RPT // 32, zr, 0)
            plsc.subcore_barrier()
            for rr in range(NW // NS):
                r = s * (NW // NS) + rr
                pltpu.sync_copy(cnts.at[r], cntv)
                nblk = jnp.max(jnp.where(pos == b, cntv[...], 0)) * (FB // EBLK)

                def edge_descs(j, p, b=b, r=r):
                    w = pl.ds(j * EBLK, EBLK)
                    return (
                        pltpu.make_async_copy(bsrc.at[b, r, w], esrcs[p],
                                              sem_e[p]),
                        pltpu.make_async_copy(bdst.at[b, r, w], edsts[p],
                                              sem_e[p]),
                        pltpu.make_async_copy(bval.at[b, r, w], evals[p],
                                              sem_e[p]),
                    )

                def start_edges(j, p):
                    for d in edge_descs(j, p):
                        d.start()

                def wait_edges(j, p):
                    for d in edge_descs(j, p):
                        d.wait()

                def gather_desc(p):
                    return pltpu.make_async_copy(x_h.at[esrcs[p]], rowss[p],
                                                 sem_g[p])

                def scatter_desc(p):
                    return pltpu.make_async_copy(rowss[p],
                                                 accum.at[edsts[p]], sem_s[p])

                @pl.when(nblk > 0)
                def _():
                    start_edges(0, 0)
                    start_edges(1, 1)
                    start_edges(2, 2)
                    wait_edges(0, 0)
                    gather_desc(0).start()

                def jgroup(j2, _):
                    for p in range(NBUF):
                        j = j2 * NBUF + p
                        q = (p + 1) % NBUF
                        gather_desc(p).wait()

                        @pl.when(j + 3 < nblk)
                        def _(j=j, p=p):
                            start_edges(j + 3, (p + 3) % NBUF)

                        @pl.when(j + 1 < nblk)
                        def _(j=j, q=q):
                            wait_edges(j + 1, q)

                        cond = j + 1 < nblk
                        if p < 3:
                            cond = jnp.logical_and(cond, j2 > 0)

                        @pl.when(cond)
                        def _(q=q):
                            scatter_desc(q).wait()

                        @pl.when(j + 1 < nblk)
                        def _(q=q):
                            gather_desc(q).start()

                        def scale(g, _, p=p):
                            for t in range(L):
                                row = g * L + t
                                wv = plsc.load_gather(
                                    evals[p], [jnp.full((L,), row, _i32)])
                                for h0 in range(HS):
                                    sl = pl.ds(h0 * L, L)
                                    rowss[p][row, sl] = rowss[p][row, sl] * wv
                            return 0

                        lax.fori_loop(0, EBLK // L, scale, 0)
                        scatter_desc(p).start(add=True)
                    return 0

                lax.fori_loop(0, nblk // NBUF, jgroup, 0)

                @pl.when(nblk > 0)
                def _():
                    for q in (1, 2, 3):
                        scatter_desc(q).wait()

            plsc.subcore_barrier()
            if mode == "plain":
                pltpu.sync_copy(accum.at[pl.ds(s * RPT, RPT)],
                                out.at[pl.ds(b * CHUNK + s * RPT, RPT)])
            else:
                yoff = NU if mode == "items" else 0

                def ep(t, _, b=b):
                    lo = s * RPT + t * 32
                    glob = b * CHUNK + lo
                    pltpu.sync_copy(accum.at[pl.ds(lo, 32)], abuf)
                    pltpu.sync_copy(x_h.at[pl.ds(yoff + glob, 32)], ybuf)
                    if mode == "items":
                        pltpu.sync_copy(deg_h.at[pl.ds(glob, 32)], dbuf)
                    for row in range(32):
                        if mode == "users":
                            for h0 in range(HS):
                                sl = pl.ds(h0 * L, L)
                                abuf[row, sl] = (abuf[row, sl]
                                                 + ybuf[row, sl]) * 0.5
                        else:
                            dv = plsc.load_gather(
                                dbuf, [jnp.full((L,), row, _i32)])
                            sw = CONV / (CONV + jnp.exp(dv * (1.0 / CONV)))
                            for h0 in range(HS):
                                sl = pl.ds(h0 * L, L)
                                abuf[row, sl] = (ybuf[row, sl]
                                                 + sw * abuf[row, sl])
                    pltpu.sync_copy(abuf, out.at[pl.ds(glob, 32)])
                    return 0

                if mode == "items":
                    @pl.when(b * CHUNK < NI)
                    def _(b=b):
                        lax.fori_loop(0, RPT // 32, ep, 0)
                else:
                    lax.fori_loop(0, RPT // 32, ep, 0)
            plsc.subcore_barrier()

    return pl.kernel(body, out_type=out_type, mesh=_mesh(),
                     compiler_params=_SC_PARAMS,
                     scratch_types=scratch, interpret=_INTERPRET)


def _make_bpr():
    G = 16
    out_type = (
        jax.ShapeDtypeStruct((BATCH, L), _f32),
        jax.ShapeDtypeStruct((BATCH, L), _f32),
        jax.ShapeDtypeStruct((NW, L), _f32),
    )
    scratch = [
        pltpu.VMEM((G,), _i32),
        pltpu.VMEM((G,), _i32),
        pltpu.VMEM((G * N_NEG,), _i32),
        pltpu.VMEM((G, H), _f32),
        pltpu.VMEM((G, H), _f32),
        pltpu.VMEM((G, H), _f32),
        pltpu.VMEM((G, H), _f32),
        pltpu.VMEM((G * N_NEG, H), _f32),
        pltpu.VMEM((G * N_NEG, H), _f32),
        pltpu.VMEM((G, L), _f32),
        pltpu.VMEM((G, L), _f32),
        pltpu.VMEM((L,), _f32),
        pltpu.SemaphoreType.DMA,
    ]

    def body(uf, itf, ue_h, ie_h, bu, bp, bn, pos_o, neg_o, reg_o,
             biu, bip, binn, xuf, xue, xitp, xiep, xitn, xien, spos, sneg,
             regv, sem):
        w = lax.axis_index("s") * NC + lax.axis_index("c")
        per_w = BATCH // NW

        def g_body(g, racc):
            e0 = w * per_w + g * G
            pltpu.sync_copy(bu.at[pl.ds(e0, G)], biu)
            pltpu.sync_copy(bp.at[pl.ds(e0, G)], bip)
            pltpu.sync_copy(bn.at[pl.ds(e0 * N_NEG, G * N_NEG)], binn)
            pltpu.async_copy(uf.at[biu], xuf, sem).wait()
            pltpu.async_copy(ue_h.at[biu], xue, sem).wait()
            pltpu.async_copy(itf.at[bip], xitp, sem).wait()
            pltpu.async_copy(ie_h.at[bip], xiep, sem).wait()
            pltpu.async_copy(itf.at[binn], xitn, sem).wait()
            pltpu.async_copy(ie_h.at[binn], xien, sem).wait()
            for e in range(G):
                pacc = jnp.zeros((L,), _f32)
                nacc = jnp.zeros((L,), _f32)
                for h0 in range(HS):
                    sl = pl.ds(h0 * L, L)
                    uv = xuf[e, sl]
                    pacc = pacc + uv * xitp[e, sl]
                    ev = xue[e, sl]
                    pv = xiep[e, sl]
                    racc = racc + ev * ev + pv * pv
                    for jn in range(N_NEG):
                        nacc = nacc + uv * xitn[e * N_NEG + jn, sl]
                        nv = xien[e * N_NEG + jn, sl]
                        racc = racc + nv * nv
                spos[e, pl.ds(0, L)] = pacc
                sneg[e, pl.ds(0, L)] = nacc
            pltpu.sync_copy(spos, pos_o.at[pl.ds(e0, G)])
            pltpu.sync_copy(sneg, neg_o.at[pl.ds(e0, G)])
            return racc

        racc = lax.fori_loop(0, per_w // G, g_body, jnp.zeros((L,), _f32))
        regv[...] = racc
        pltpu.sync_copy(regv, reg_o.at[w])

    return pl.kernel(body, out_type=out_type, mesh=_mesh(),
                     compiler_params=_SC_PARAMS,
                     scratch_types=scratch, interpret=_INTERPRET)


def _tc_final(pos_ref, neg_ref, reg_ref, loss_ref, regl_ref):
    pos = jnp.sum(pos_ref[...], axis=1)
    neg = jnp.sum(neg_ref[...], axis=1) * (1.0 / N_NEG)
    d = neg - pos
    sp = jnp.maximum(d, 0.0) + jnp.log1p(jnp.exp(-jnp.abs(d)))
    loss_ref[...] = jnp.mean(sp).reshape(1, 1)
    regl_ref[...] = (0.5 * jnp.sum(reg_ref[...]) / BATCH).reshape(1, 1)


def _pad_edges(dst, src, val, nblk_in, spread_mod):
    et = nblk_in * FB
    n = dst.shape[0]
    padn = NW * et - n
    ar = jnp.arange(padn, dtype=_i32)
    sp = ar % spread_mod
    dst = jnp.concatenate([dst.astype(_i32), sp])
    src = jnp.concatenate([src.astype(_i32), sp])
    val = jnp.concatenate([val.astype(_f32), jnp.zeros((padn,), _f32)])
    return (dst.reshape(NW, nblk_in, FB), src.reshape(NW, nblk_in, FB),
            val.reshape(NW, nblk_in, FB))


def kernel(user_emb, item_emb, a_indices, a_values, s_indices, s_values,
           item_r, item_c, enhance_weight, item_degree,
           batch_user, batch_pos, batch_neg):
    x0 = jnp.concatenate(
        [user_emb.astype(_f32), item_emb.astype(_f32),
         jnp.zeros((NT_PAD - NT, H), _f32)], axis=0)
    nbi_a = -(-(1000000 // NW) // FB)
    nbi_s = -(-(500000 // NW) // FB)
    nbi_i = -(-(200000 // NW) // FB)

    ad, asx, av = _pad_edges(a_indices[0], a_indices[1], a_values, nbi_a,
                             CHUNK)
    bs_a, bd_a, bv_a, cn_a = _make_prepass(NB_A, nbi_a, 0)(ad, asx, av)
    spmm_a = _make_spmm(NB_A, "plain")
    y1 = spmm_a(x0, bs_a, bd_a, bv_a, cn_a)
    y2 = spmm_a(y1, bs_a, bd_a, bv_a, cn_a)

    sd, ssx, sv = _pad_edges(s_indices[0], s_indices[1], s_values, nbi_s,
                             CHUNK)
    bs_s, bd_s, bv_s, cn_s = _make_prepass(NB_S, nbi_s, 0)(sd, ssx, sv)
    uf = _make_spmm(NB_S, "users")(y2, bs_s, bd_s, bv_s, cn_s)

    idd, isx, iv = _pad_edges(item_r, item_c, enhance_weight, nbi_i, CHUNK)
    bs_i, bd_i, bv_i, cn_i = _make_prepass(NB_S, nbi_i, NU)(idd, isx, iv)
    degp = jnp.concatenate(
        [item_degree.astype(_f32), jnp.zeros((NU_PAD - NI,), _f32)])
    itf = _make_spmm(NB_S, "items")(y2, bs_i, bd_i, bv_i, cn_i, degp)

    pos_p, neg_p, reg_p = _make_bpr()(
        uf, itf, user_emb.astype(_f32), item_emb.astype(_f32),
        batch_user.astype(_i32), batch_pos.astype(_i32),
        batch_neg.reshape(-1).astype(_i32))
    outs = pl.pallas_call(
        _tc_final,
        out_shape=(jax.ShapeDtypeStruct((1, 1), _f32),
                   jax.ShapeDtypeStruct((1, 1), _f32)),
        interpret=_INTERPRET,
    )(pos_p, neg_p, reg_p)
    return (outs[0][0, 0], outs[1][0, 0])


# speedup vs baseline: 5.9130x; 1.5635x over previous
import jax
import jax.numpy as jnp
from jax import lax
from jax.experimental import pallas as pl
from jax.experimental.pallas import tpu as pltpu
from jax.experimental.pallas import tpu_sc as plsc

NU = 50000
NI = 50000
NT = 100000
H = 64
L = 16
HS = H // L
NC = 2
NS = 16
NW = NC * NS
CHUNK = 16896
RPT = CHUNK // NS
NB_A = 6
NB_S = 4
NT_PAD = NB_A * CHUNK
NU_PAD = NB_S * CHUNK
FB = 512
EBLK = 128
BATCH = 4096
N_NEG = 10
CONV = 10.0
_INTERPRET = False

_f32 = jnp.float32
_i32 = jnp.int32
_SC_PARAMS = pltpu.CompilerParams(needs_layout_passes=False,
                                  use_tc_tiling_on_sc=False)


def _mesh():
    return plsc.VectorSubcoreMesh(core_axis_name="c", subcore_axis_name="s",
                                  num_cores=NC, num_subcores=NS)


def _scal(x):
    x = jnp.asarray(x)
    return jnp.max(x) if x.ndim else x


def _lanes():
    return lax.broadcasted_iota(_i32, (L,), 0)


def _make_prepass(nb, nblk_in, src_off):
    cap = (nblk_in + 1) * FB
    out_type = (
        jax.ShapeDtypeStruct((nb, NW, cap), _i32),
        jax.ShapeDtypeStruct((nb, NW, cap), _i32),
        jax.ShapeDtypeStruct((nb, NW, cap), _f32),
        jax.ShapeDtypeStruct((NW, L), _i32),
    )
    scratch = [
        pltpu.VMEM((FB,), _i32),
        pltpu.VMEM((FB,), _i32),
        pltpu.VMEM((FB,), _f32),
    ]
    scratch += [pltpu.VMEM((2 * FB,), _i32) for _ in range(nb)]
    scratch += [pltpu.VMEM((2 * FB,), _i32) for _ in range(nb)]
    scratch += [pltpu.VMEM((2 * FB,), _f32) for _ in range(nb)]
    scratch.append(pltpu.VMEM((L,), _i32))

    def body(dst_h, src_h, val_h, bsrc, bdst, bval, cnts, ind, ins, inv, *rest):
        stg_s = rest[0:nb]
        stg_d = rest[nb:2 * nb]
        stg_v = rest[2 * nb:3 * nb]
        cntv = rest[3 * nb]
        w = lax.axis_index("s") * NC + lax.axis_index("c")
        pos = _lanes()

        def flush(b, nf):
            pltpu.sync_copy(stg_s[b].at[pl.ds(0, FB)],
                            bsrc.at[b, w, pl.ds(nf * FB, FB)])
            pltpu.sync_copy(stg_d[b].at[pl.ds(0, FB)],
                            bdst.at[b, w, pl.ds(nf * FB, FB)])
            pltpu.sync_copy(stg_v[b].at[pl.ds(0, FB)],
                            bval.at[b, w, pl.ds(nf * FB, FB)])

        def blk(j, carry):
            ptrs = list(carry[:nb])
            nfs = list(carry[nb:])
            pltpu.sync_copy(dst_h.at[w, j], ind)
            pltpu.sync_copy(src_h.at[w, j], ins)
            pltpu.sync_copy(val_h.at[w, j], inv)
            for v in range(FB // L):
                sl = pl.ds(v * L, L)
                dv = ind[sl]
                sv = ins[sl] + src_off if src_off else ins[sl]
                vv = inv[sl]
                bv = lax.div(dv, jnp.int32(CHUNK))
                dl = dv - bv * CHUNK
                ms = [bv == b for b in range(nb)]
                incls = [plsc.cumsum(m.astype(_i32)) for m in ms]
                for b in range(nb):
                    tgt = ptrs[b] + incls[b] - 1
                    plsc.store_scatter(stg_s[b], [tgt], sv, mask=ms[b])
                    plsc.store_scatter(stg_d[b], [tgt], dl, mask=ms[b])
                    plsc.store_scatter(stg_v[b], [tgt], vv, mask=ms[b])
                    ptrs[b] = ptrs[b] + jnp.max(incls[b])
            for b in range(nb):
                fl = ptrs[b] >= FB

                @pl.when(fl)
                def _(b=b, nf=nfs[b]):
                    flush(b, nf)
                    for v in range(FB // L):
                        lo = pl.ds(v * L, L)
                        hi = pl.ds(FB + v * L, L)
                        stg_s[b][lo] = stg_s[b][hi]
                        stg_d[b][lo] = stg_d[b][hi]
                        stg_v[b][lo] = stg_v[b][hi]

                ptrs[b] = jnp.where(fl, ptrs[b] - FB, ptrs[b])
                nfs[b] = jnp.where(fl, nfs[b] + 1, nfs[b])
            return tuple(ptrs) + tuple(nfs)

        init = tuple(jnp.int32(0) for _ in range(2 * nb))
        carry = lax.fori_loop(0, nblk_in, blk, init)
        ptrs = carry[:nb]
        nfs = carry[nb:]
        cv = jnp.zeros((L,), _i32)
        for b in range(nb):
            ptr = ptrs[b]
            for v in range(FB // L):
                sl = pl.ds(v * L, L)
                gpos = pos + v * L
                mi = (gpos >= ptr).astype(_i32)
                mf = mi.astype(_f32)
                stg_v[b][sl] = stg_v[b][sl] * (1.0 - mf)
                stg_d[b][sl] = stg_d[b][sl] * (1 - mi) + gpos * mi
                stg_s[b][sl] = stg_s[b][sl] * (1 - mi) + gpos * mi
            flush(b, nfs[b])
            cv = jnp.where(pos == b,
                           jnp.where(ptr > 0, nfs[b] + 1, nfs[b]), cv)
        cntv[...] = cv
        pltpu.sync_copy(cntv, cnts.at[w])

    return pl.kernel(body, out_type=out_type, mesh=_mesh(),
                     compiler_params=_SC_PARAMS,
                     scratch_types=scratch, interpret=_INTERPRET)


def _make_spmm(nb, mode):
    nbs = nb // NC
    NBUF = 4
    out_type = jax.ShapeDtypeStruct((nb * CHUNK, H), _f32)
    scratch = [pltpu.VMEM_SHARED((CHUNK, H), _f32)]
    scratch += [pltpu.VMEM((EBLK,), _i32) for _ in range(NBUF)]
    scratch += [pltpu.VMEM((EBLK,), _i32) for _ in range(NBUF)]
    scratch += [pltpu.VMEM((EBLK,), _f32) for _ in range(NBUF)]
    scratch += [pltpu.VMEM((EBLK, H), _f32) for _ in range(NBUF)]
    scratch += [pltpu.VMEM((32, H), _f32), pltpu.VMEM((L,), _i32)]
    scratch += [pltpu.SemaphoreType.DMA for _ in range(3 * NBUF)]
    if mode != "plain":
        scratch += [pltpu.VMEM((32, H), _f32), pltpu.VMEM((32, H), _f32)]
    if mode == "items":
        scratch.append(pltpu.VMEM((32,), _f32))

    def body(*args):
        x_h, bsrc, bdst, bval, cnts = args[:5]
        args = args[5:]
        if mode == "items":
            deg_h = args[0]
            args = args[1:]
        out = args[0]
        args = args[1:]
        accum = args[0]
        esrcs = args[1:1 + NBUF]
        edsts = args[1 + NBUF:1 + 2 * NBUF]
        evals = args[1 + 2 * NBUF:1 + 3 * NBUF]
        rowss = args[1 + 3 * NBUF:1 + 4 * NBUF]
        zbuf = args[1 + 4 * NBUF]
        cntv = args[2 + 4 * NBUF]
        base = 3 + 4 * NBUF
        sem_e = args[base:base + NBUF]
        sem_g = args[base + NBUF:base + 2 * NBUF]
        sem_s = args[base + 2 * NBUF:base + 3 * NBUF]
        rest = args[base + 3 * NBUF:]
        if mode != "plain":
            abuf, ybuf = rest[0], rest[1]
        if mode == "items":
            dbuf = rest[2]
        c = lax.axis_index("c")
        s = lax.axis_index("s")
        pos = _lanes()
        zero = jnp.zeros((L,), _f32)
        for r0 in range(32):
            for h0 in range(HS):
                zbuf[r0, pl.ds(h0 * L, L)] = zero
        for k in range(nbs):
            b = c * nbs + k

            def zr(t, _):
                pltpu.sync_copy(zbuf, accum.at[pl.ds(s * RPT + t * 32, 32)])
                return 0

            lax.fori_loop(0, RPT // 32, zr, 0)
            plsc.subcore_barrier()
            for rr in range(NW // NS):
                r = s * (NW // NS) + rr
                pltpu.sync_copy(cnts.at[r], cntv)
                nblk = jnp.max(jnp.where(pos == b, cntv[...], 0)) * (FB // EBLK)

                def edge_descs(j, p, b=b, r=r):
                    w = pl.ds(j * EBLK, EBLK)
                    return (
                        pltpu.make_async_copy(bsrc.at[b, r, w], esrcs[p],
                                              sem_e[p]),
                        pltpu.make_async_copy(bdst.at[b, r, w], edsts[p],
                                              sem_e[p]),
                        pltpu.make_async_copy(bval.at[b, r, w], evals[p],
                                              sem_e[p]),
                    )

                def start_edges(j, p):
                    for d in edge_descs(j, p):
                        d.start()

                def wait_edges(j, p):
                    for d in edge_descs(j, p):
                        d.wait()

                def gather_desc(p):
                    return pltpu.make_async_copy(x_h.at[esrcs[p]], rowss[p],
                                                 sem_g[p])

                def scatter_desc(p):
                    return pltpu.make_async_copy(rowss[p],
                                                 accum.at[edsts[p]], sem_s[p])

                @pl.when(nblk > 0)
                def _():
                    start_edges(0, 0)
                    start_edges(1, 1)
                    start_edges(2, 2)
                    wait_edges(0, 0)
                    gather_desc(0).start()

                def jgroup(j2, _):
                    for p in range(NBUF):
                        j = j2 * NBUF + p
                        q = (p + 1) % NBUF
                        gather_desc(p).wait()

                        @pl.when(j + 3 < nblk)
                        def _(j=j, p=p):
                            start_edges(j + 3, (p + 3) % NBUF)

                        @pl.when(j + 1 < nblk)
                        def _(j=j, q=q):
                            wait_edges(j + 1, q)

                        cond = j + 1 < nblk
                        if p < 3:
                            cond = jnp.logical_and(cond, j2 > 0)

                        @pl.when(cond)
                        def _(q=q):
                            scatter_desc(q).wait()

                        @pl.when(j + 1 < nblk)
                        def _(q=q):
                            gather_desc(q).start()

                        def scale(g, p=p):
                            for t in range(L):
                                row = g * L + t
                                wv = plsc.load_gather(
                                    evals[p], [jnp.full((L,), row, _i32)])
                                vals = [rowss[p][row, pl.ds(h0 * L, L)]
                                        for h0 in range(HS)]
                                for h0 in range(HS):
                                    rowss[p][row, pl.ds(h0 * L, L)] = (
                                        vals[h0] * wv)

                        plsc.parallel_loop(0, EBLK // L, unroll=2)(scale)
                        scatter_desc(p).start(add=True)
                    return 0

                lax.fori_loop(0, nblk // NBUF, jgroup, 0)

                @pl.when(nblk > 0)
                def _():
                    for q in (1, 2, 3):
                        scatter_desc(q).wait()

            plsc.subcore_barrier()
            if mode == "plain":
                pltpu.sync_copy(accum.at[pl.ds(s * RPT, RPT)],
                                out.at[pl.ds(b * CHUNK + s * RPT, RPT)])
            else:
                yoff = NU if mode == "items" else 0

                def ep(t, _, b=b):
                    lo = s * RPT + t * 32
                    glob = b * CHUNK + lo
                    pltpu.sync_copy(accum.at[pl.ds(lo, 32)], abuf)
                    pltpu.sync_copy(x_h.at[pl.ds(yoff + glob, 32)], ybuf)
                    if mode == "items":
                        pltpu.sync_copy(deg_h.at[pl.ds(glob, 32)], dbuf)
                    for row in range(32):
                        if mode == "users":
                            for h0 in range(HS):
                                sl = pl.ds(h0 * L, L)
                                abuf[row, sl] = (abuf[row, sl]
                                                 + ybuf[row, sl]) * 0.5
                        else:
                            dv = plsc.load_gather(
                                dbuf, [jnp.full((L,), row, _i32)])
                            sw = CONV / (CONV + jnp.exp(dv * (1.0 / CONV)))
                            for h0 in range(HS):
                                sl = pl.ds(h0 * L, L)
                                abuf[row, sl] = (ybuf[row, sl]
                                                 + sw * abuf[row, sl])
                    pltpu.sync_copy(abuf, out.at[pl.ds(glob, 32)])
                    return 0

                if mode == "items":
                    @pl.when(b * CHUNK < NI)
                    def _(b=b):
                        lax.fori_loop(0, RPT // 32, ep, 0)
                else:
                    lax.fori_loop(0, RPT // 32, ep, 0)
            plsc.subcore_barrier()

    return pl.kernel(body, out_type=out_type, mesh=_mesh(),
                     compiler_params=_SC_PARAMS,
                     scratch_types=scratch, interpret=_INTERPRET)


def _make_bpr():
    G = 16
    out_type = (
        jax.ShapeDtypeStruct((BATCH, L), _f32),
        jax.ShapeDtypeStruct((BATCH, L), _f32),
        jax.ShapeDtypeStruct((NW, L), _f32),
    )
    scratch = [
        pltpu.VMEM((G,), _i32),
        pltpu.VMEM((G,), _i32),
        pltpu.VMEM((G * N_NEG,), _i32),
        pltpu.VMEM((G, H), _f32),
        pltpu.VMEM((G, H), _f32),
        pltpu.VMEM((G, H), _f32),
        pltpu.VMEM((G, H), _f32),
        pltpu.VMEM((G * N_NEG, H), _f32),
        pltpu.VMEM((G * N_NEG, H), _f32),
        pltpu.VMEM((G, L), _f32),
        pltpu.VMEM((G, L), _f32),
        pltpu.VMEM((L,), _f32),
        pltpu.SemaphoreType.DMA,
    ]

    def body(uf, itf, ue_h, ie_h, bu, bp, bn, pos_o, neg_o, reg_o,
             biu, bip, binn, xuf, xue, xitp, xiep, xitn, xien, spos, sneg,
             regv, sem):
        w = lax.axis_index("s") * NC + lax.axis_index("c")
        per_w = BATCH // NW

        def g_body(g, racc):
            e0 = w * per_w + g * G
            di = (pltpu.async_copy(bu.at[pl.ds(e0, G)], biu, sem),
                  pltpu.async_copy(bp.at[pl.ds(e0, G)], bip, sem),
                  pltpu.async_copy(bn.at[pl.ds(e0 * N_NEG, G * N_NEG)], binn,
                                   sem))
            for d in di:
                d.wait()
            dg = (pltpu.async_copy(uf.at[biu], xuf, sem),
                  pltpu.async_copy(ue_h.at[biu], xue, sem),
                  pltpu.async_copy(itf.at[bip], xitp, sem),
                  pltpu.async_copy(ie_h.at[bip], xiep, sem),
                  pltpu.async_copy(itf.at[binn], xitn, sem),
                  pltpu.async_copy(ie_h.at[binn], xien, sem))
            for d in dg:
                d.wait()
            for e in range(G):
                pacc = jnp.zeros((L,), _f32)
                nacc = jnp.zeros((L,), _f32)
                for h0 in range(HS):
                    sl = pl.ds(h0 * L, L)
                    uv = xuf[e, sl]
                    pacc = pacc + uv * xitp[e, sl]
                    ev = xue[e, sl]
                    pv = xiep[e, sl]
                    racc = racc + ev * ev + pv * pv
                    for jn in range(N_NEG):
                        nacc = nacc + uv * xitn[e * N_NEG + jn, sl]
                        nv = xien[e * N_NEG + jn, sl]
                        racc = racc + nv * nv
                spos[e, pl.ds(0, L)] = pacc
                sneg[e, pl.ds(0, L)] = nacc
            pltpu.sync_copy(spos, pos_o.at[pl.ds(e0, G)])
            pltpu.sync_copy(sneg, neg_o.at[pl.ds(e0, G)])
            return racc

        racc = lax.fori_loop(0, per_w // G, g_body, jnp.zeros((L,), _f32))
        regv[...] = racc
        pltpu.sync_copy(regv, reg_o.at[w])

    return pl.kernel(body, out_type=out_type, mesh=_mesh(),
                     compiler_params=_SC_PARAMS,
                     scratch_types=scratch, interpret=_INTERPRET)


def _tc_final(pos_ref, neg_ref, reg_ref, loss_ref, regl_ref):
    pos = jnp.sum(pos_ref[...], axis=1)
    neg = jnp.sum(neg_ref[...], axis=1) * (1.0 / N_NEG)
    d = neg - pos
    sp = jnp.maximum(d, 0.0) + jnp.log1p(jnp.exp(-jnp.abs(d)))
    loss_ref[...] = jnp.mean(sp).reshape(1, 1)
    regl_ref[...] = (0.5 * jnp.sum(reg_ref[...]) / BATCH).reshape(1, 1)


def _pad_edges(dst, src, val, nblk_in, spread_mod):
    et = nblk_in * FB
    n = dst.shape[0]
    padn = NW * et - n
    ar = jnp.arange(padn, dtype=_i32)
    sp = ar % spread_mod
    dst = jnp.concatenate([dst.astype(_i32), sp])
    src = jnp.concatenate([src.astype(_i32), sp])
    val = jnp.concatenate([val.astype(_f32), jnp.zeros((padn,), _f32)])
    return (dst.reshape(NW, nblk_in, FB), src.reshape(NW, nblk_in, FB),
            val.reshape(NW, nblk_in, FB))


def kernel(user_emb, item_emb, a_indices, a_values, s_indices, s_values,
           item_r, item_c, enhance_weight, item_degree,
           batch_user, batch_pos, batch_neg):
    x0 = jnp.concatenate(
        [user_emb.astype(_f32), item_emb.astype(_f32),
         jnp.zeros((NT_PAD - NT, H), _f32)], axis=0)
    nbi_a = -(-(1000000 // NW) // FB)
    nbi_s = -(-(500000 // NW) // FB)
    nbi_i = -(-(200000 // NW) // FB)

    ad, asx, av = _pad_edges(a_indices[0], a_indices[1], a_values, nbi_a,
                             CHUNK)
    bs_a, bd_a, bv_a, cn_a = _make_prepass(NB_A, nbi_a, 0)(ad, asx, av)
    spmm_a = _make_spmm(NB_A, "plain")
    y1 = spmm_a(x0, bs_a, bd_a, bv_a, cn_a)
    y2 = spmm_a(y1, bs_a, bd_a, bv_a, cn_a)

    sd, ssx, sv = _pad_edges(s_indices[0], s_indices[1], s_values, nbi_s,
                             CHUNK)
    bs_s, bd_s, bv_s, cn_s = _make_prepass(NB_S, nbi_s, 0)(sd, ssx, sv)
    uf = _make_spmm(NB_S, "users")(y2, bs_s, bd_s, bv_s, cn_s)

    idd, isx, iv = _pad_edges(item_r, item_c, enhance_weight, nbi_i, CHUNK)
    bs_i, bd_i, bv_i, cn_i = _make_prepass(NB_S, nbi_i, NU)(idd, isx, iv)
    degp = jnp.concatenate(
        [item_degree.astype(_f32), jnp.zeros((NU_PAD - NI,), _f32)])
    itf = _make_spmm(NB_S, "items")(y2, bs_i, bd_i, bv_i, cn_i, degp)

    pos_p, neg_p, reg_p = _make_bpr()(
        uf, itf, user_emb.astype(_f32), item_emb.astype(_f32),
        batch_user.astype(_i32), batch_pos.astype(_i32),
        batch_neg.reshape(-1).astype(_i32))
    outs = pl.pallas_call(
        _tc_final,
        out_shape=(jax.ShapeDtypeStruct((1, 1), _f32),
                   jax.ShapeDtypeStruct((1, 1), _f32)),
        interpret=_INTERPRET,
    )(pos_p, neg_p, reg_p)
    return (outs[0][0, 0], outs[1][0, 0])


# speedup vs baseline: 6.0807x; 1.0284x over previous
import jax
import jax.numpy as jnp
from jax import lax
from jax.experimental import pallas as pl
from jax.experimental.pallas import tpu as pltpu
from jax.experimental.pallas import tpu_sc as plsc

NU = 50000
NI = 50000
NT = 100000
H = 64
L = 16
HS = H // L
NC = 2
NS = 16
NW = NC * NS
CHUNK = 16896
RPT = CHUNK // NS
NB_A = 6
NB_S = 4
NT_PAD = NB_A * CHUNK
NU_PAD = NB_S * CHUNK
FB = 512
EBLK = 128
BATCH = 4096
N_NEG = 10
CONV = 10.0
_INTERPRET = False

_f32 = jnp.float32
_i32 = jnp.int32
_SC_PARAMS = pltpu.CompilerParams(needs_layout_passes=False,
                                  use_tc_tiling_on_sc=False)


def _mesh():
    return plsc.VectorSubcoreMesh(core_axis_name="c", subcore_axis_name="s",
                                  num_cores=NC, num_subcores=NS)


def _scal(x):
    x = jnp.asarray(x)
    return jnp.max(x) if x.ndim else x


def _lanes():
    return lax.broadcasted_iota(_i32, (L,), 0)


def _make_prepass(nb, nblk_in, src_off):
    cap = (nblk_in + 1) * FB
    out_type = (
        jax.ShapeDtypeStruct((nb, NW, cap), _i32),
        jax.ShapeDtypeStruct((nb, NW, cap), _i32),
        jax.ShapeDtypeStruct((nb, NW, cap), _f32),
        jax.ShapeDtypeStruct((NW, L), _i32),
    )
    scratch = [
        pltpu.VMEM((FB,), _i32),
        pltpu.VMEM((FB,), _i32),
        pltpu.VMEM((FB,), _f32),
    ]
    scratch += [pltpu.VMEM((2 * FB,), _i32) for _ in range(nb)]
    scratch += [pltpu.VMEM((2 * FB,), _i32) for _ in range(nb)]
    scratch += [pltpu.VMEM((2 * FB,), _f32) for _ in range(nb)]
    scratch.append(pltpu.VMEM((L,), _i32))

    def body(dst_h, src_h, val_h, bsrc, bdst, bval, cnts, ind, ins, inv, *rest):
        stg_s = rest[0:nb]
        stg_d = rest[nb:2 * nb]
        stg_v = rest[2 * nb:3 * nb]
        cntv = rest[3 * nb]
        w = lax.axis_index("s") * NC + lax.axis_index("c")
        pos = _lanes()

        def flush(b, nf):
            pltpu.sync_copy(stg_s[b].at[pl.ds(0, FB)],
                            bsrc.at[b, w, pl.ds(nf * FB, FB)])
            pltpu.sync_copy(stg_d[b].at[pl.ds(0, FB)],
                            bdst.at[b, w, pl.ds(nf * FB, FB)])
            pltpu.sync_copy(stg_v[b].at[pl.ds(0, FB)],
                            bval.at[b, w, pl.ds(nf * FB, FB)])

        def blk(j, carry):
            ptrs = list(carry[:nb])
            nfs = list(carry[nb:])
            pltpu.sync_copy(dst_h.at[w, j], ind)
            pltpu.sync_copy(src_h.at[w, j], ins)
            pltpu.sync_copy(val_h.at[w, j], inv)
            for v in range(FB // L):
                sl = pl.ds(v * L, L)
                dv = ind[sl]
                sv = ins[sl] + src_off if src_off else ins[sl]
                vv = inv[sl]
                bv = lax.div(dv, jnp.int32(CHUNK))
                dl = dv - bv * CHUNK
                ms = [bv == b for b in range(nb)]
                incls = [plsc.cumsum(m.astype(_i32)) for m in ms]
                cnts_v = [plsc.all_reduce_population_count(m) for m in ms]
                for b in range(nb):
                    tgt = ptrs[b] + incls[b] - 1
                    plsc.store_scatter(stg_s[b], [tgt], sv, mask=ms[b])
                    plsc.store_scatter(stg_d[b], [tgt], dl, mask=ms[b])
                    plsc.store_scatter(stg_v[b], [tgt], vv, mask=ms[b])
                    ptrs[b] = ptrs[b] + cnts_v[b]
            for b in range(nb):
                ptr_s = jnp.max(ptrs[b])
                fl = ptr_s >= FB

                @pl.when(fl)
                def _(b=b, nf=nfs[b]):
                    flush(b, nf)
                    for v in range(FB // L):
                        lo = pl.ds(v * L, L)
                        hi = pl.ds(FB + v * L, L)
                        stg_s[b][lo] = stg_s[b][hi]
                        stg_d[b][lo] = stg_d[b][hi]
                        stg_v[b][lo] = stg_v[b][hi]

                ptrs[b] = jnp.where(fl, ptrs[b] - FB, ptrs[b])
                nfs[b] = jnp.where(fl, nfs[b] + 1, nfs[b])
            return tuple(ptrs) + tuple(nfs)

        init = tuple(jnp.zeros((L,), _i32) for _ in range(nb)) + tuple(
            jnp.int32(0) for _ in range(nb))
        carry = lax.fori_loop(0, nblk_in, blk, init)
        ptrs = [jnp.max(p) for p in carry[:nb]]
        nfs = carry[nb:]
        cv = jnp.zeros((L,), _i32)
        for b in range(nb):
            ptr = ptrs[b]
            for v in range(FB // L):
                sl = pl.ds(v * L, L)
                gpos = pos + v * L
                mi = (gpos >= ptr).astype(_i32)
                mf = mi.astype(_f32)
                stg_v[b][sl] = stg_v[b][sl] * (1.0 - mf)
                stg_d[b][sl] = stg_d[b][sl] * (1 - mi) + gpos * mi
                stg_s[b][sl] = stg_s[b][sl] * (1 - mi) + gpos * mi
            flush(b, nfs[b])
            cv = jnp.where(pos == b,
                           jnp.where(ptr > 0, nfs[b] + 1, nfs[b]), cv)
        cntv[...] = cv
        pltpu.sync_copy(cntv, cnts.at[w])

    return pl.kernel(body, out_type=out_type, mesh=_mesh(),
                     compiler_params=_SC_PARAMS,
                     scratch_types=scratch, interpret=_INTERPRET)


def _make_spmm(nb, mode):
    nbs = nb // NC
    NBUF = 4
    out_type = jax.ShapeDtypeStruct((nb * CHUNK, H), _f32)
    scratch = [pltpu.VMEM_SHARED((CHUNK, H), _f32)]
    scratch += [pltpu.VMEM((EBLK,), _i32) for _ in range(NBUF)]
    scratch += [pltpu.VMEM((EBLK,), _i32) for _ in range(NBUF)]
    scratch += [pltpu.VMEM((EBLK,), _f32) for _ in range(NBUF)]
    scratch += [pltpu.VMEM((EBLK, H), _f32) for _ in range(NBUF)]
    scratch += [pltpu.VMEM((32, H), _f32), pltpu.VMEM((L,), _i32)]
    scratch += [pltpu.SemaphoreType.DMA for _ in range(3 * NBUF)]
    if mode != "plain":
        scratch += [pltpu.VMEM((32, H), _f32), pltpu.VMEM((32, H), _f32)]
    if mode == "items":
        scratch.append(pltpu.VMEM((32,), _f32))

    def body(*args):
        x_h, bsrc, bdst, bval, cnts = args[:5]
        args = args[5:]
        if mode == "items":
            deg_h = args[0]
            args = args[1:]
        out = args[0]
        args = args[1:]
        accum = args[0]
        esrcs = args[1:1 + NBUF]
        edsts = args[1 + NBUF:1 + 2 * NBUF]
        evals = args[1 + 2 * NBUF:1 + 3 * NBUF]
        rowss = args[1 + 3 * NBUF:1 + 4 * NBUF]
        zbuf = args[1 + 4 * NBUF]
        cntv = args[2 + 4 * NBUF]
        base = 3 + 4 * NBUF
        sem_e = args[base:base + NBUF]
        sem_g = args[base + NBUF:base + 2 * NBUF]
        sem_s = args[base + 2 * NBUF:base + 3 * NBUF]
        rest = args[base + 3 * NBUF:]
        if mode != "plain":
            abuf, ybuf = rest[0], rest[1]
        if mode == "items":
            dbuf = rest[2]
        c = lax.axis_index("c")
        s = lax.axis_index("s")
        pos = _lanes()
        zero = jnp.zeros((L,), _f32)
        for r0 in range(32):
            for h0 in range(HS):
                zbuf[r0, pl.ds(h0 * L, L)] = zero
        for k in range(nbs):
            b = c * nbs + k

            def zr(t, _):
                pltpu.sync_copy(zbuf, accum.at[pl.ds(s * RPT + t * 32, 32)])
                return 0

            lax.fori_loop(0, RPT // 32, zr, 0)
            plsc.subcore_barrier()
            for rr in range(NW // NS):
                r = s * (NW // NS) + rr
                pltpu.sync_copy(cnts.at[r], cntv)
                nblk = jnp.max(jnp.where(pos == b, cntv[...], 0)) * (FB // EBLK)

                def edge_descs(j, p, b=b, r=r):
                    w = pl.ds(j * EBLK, EBLK)
                    return (
                        pltpu.make_async_copy(bsrc.at[b, r, w], esrcs[p],
                                              sem_e[p]),
                        pltpu.make_async_copy(bdst.at[b, r, w], edsts[p],
                                              sem_e[p]),
                        pltpu.make_async_copy(bval.at[b, r, w], evals[p],
                                              sem_e[p]),
                    )

                def start_edges(j, p):
                    for d in edge_descs(j, p):
                        d.start()

                def wait_edges(j, p):
                    for d in edge_descs(j, p):
                        d.wait()

                def gather_desc(p):
                    return pltpu.make_async_copy(x_h.at[esrcs[p]], rowss[p],
                                                 sem_g[p])

                def scatter_desc(p):
                    return pltpu.make_async_copy(rowss[p],
                                                 accum.at[edsts[p]], sem_s[p])

                @pl.when(nblk > 0)
                def _():
                    start_edges(0, 0)
                    start_edges(1, 1)
                    start_edges(2, 2)
                    wait_edges(0, 0)
                    gather_desc(0).start()

                def jgroup(j2, _):
                    for p in range(NBUF):
                        j = j2 * NBUF + p
                        q = (p + 1) % NBUF
                        gather_desc(p).wait()

                        @pl.when(j + 3 < nblk)
                        def _(j=j, p=p):
                            start_edges(j + 3, (p + 3) % NBUF)

                        @pl.when(j + 1 < nblk)
                        def _(j=j, q=q):
                            wait_edges(j + 1, q)

                        cond = j + 1 < nblk
                        if p < 3:
                            cond = jnp.logical_and(cond, j2 > 0)

                        @pl.when(cond)
                        def _(q=q):
                            scatter_desc(q).wait()

                        @pl.when(j + 1 < nblk)
                        def _(q=q):
                            gather_desc(q).start()

                        def scale(g, p=p):
                            for t in range(L):
                                row = g * L + t
                                wv = plsc.load_gather(
                                    evals[p], [jnp.full((L,), row, _i32)])
                                vals = [rowss[p][row, pl.ds(h0 * L, L)]
                                        for h0 in range(HS)]
                                for h0 in range(HS):
                                    rowss[p][row, pl.ds(h0 * L, L)] = (
                                        vals[h0] * wv)

                        plsc.parallel_loop(0, EBLK // L, unroll=2)(scale)
                        scatter_desc(p).start(add=True)
                    return 0

                lax.fori_loop(0, nblk // NBUF, jgroup, 0)

                @pl.when(nblk > 0)
                def _():
                    for q in (1, 2, 3):
                        scatter_desc(q).wait()

            plsc.subcore_barrier()
            if mode == "plain":
                pltpu.sync_copy(accum.at[pl.ds(s * RPT, RPT)],
                                out.at[pl.ds(b * CHUNK + s * RPT, RPT)])
            else:
                yoff = NU if mode == "items" else 0

                def ep(t, _, b=b):
                    lo = s * RPT + t * 32
                    glob = b * CHUNK + lo
                    pltpu.sync_copy(accum.at[pl.ds(lo, 32)], abuf)
                    pltpu.sync_copy(x_h.at[pl.ds(yoff + glob, 32)], ybuf)
                    if mode == "items":
                        pltpu.sync_copy(deg_h.at[pl.ds(glob, 32)], dbuf)
                    for row in range(32):
                        if mode == "users":
                            for h0 in range(HS):
                                sl = pl.ds(h0 * L, L)
                                abuf[row, sl] = (abuf[row, sl]
                                                 + ybuf[row, sl]) * 0.5
                        else:
                            dv = plsc.load_gather(
                                dbuf, [jnp.full((L,), row, _i32)])
                            sw = CONV / (CONV + jnp.exp(dv * (1.0 / CONV)))
                            for h0 in range(HS):
                                sl = pl.ds(h0 * L, L)
                                abuf[row, sl] = (ybuf[row, sl]
                                                 + sw * abuf[row, sl])
                    pltpu.sync_copy(abuf, out.at[pl.ds(glob, 32)])
                    return 0

                if mode == "items":
                    @pl.when(b * CHUNK < NI)
                    def _(b=b):
                        lax.fori_loop(0, RPT // 32, ep, 0)
                else:
                    lax.fori_loop(0, RPT // 32, ep, 0)
            plsc.subcore_barrier()

    return pl.kernel(body, out_type=out_type, mesh=_mesh(),
                     compiler_params=_SC_PARAMS,
                     scratch_types=scratch, interpret=_INTERPRET)


def _make_bpr():
    G = 16
    out_type = (
        jax.ShapeDtypeStruct((BATCH, L), _f32),
        jax.ShapeDtypeStruct((BATCH, L), _f32),
        jax.ShapeDtypeStruct((NW, L), _f32),
    )
    scratch = [
        pltpu.VMEM((G,), _i32),
        pltpu.VMEM((G,), _i32),
        pltpu.VMEM((G * N_NEG,), _i32),
        pltpu.VMEM((G, H), _f32),
        pltpu.VMEM((G, H), _f32),
        pltpu.VMEM((G, H), _f32),
        pltpu.VMEM((G, H), _f32),
        pltpu.VMEM((G * N_NEG, H), _f32),
        pltpu.VMEM((G * N_NEG, H), _f32),
        pltpu.VMEM((G, L), _f32),
        pltpu.VMEM((G, L), _f32),
        pltpu.VMEM((L,), _f32),
        pltpu.SemaphoreType.DMA,
    ]

    def body(uf, itf, ue_h, ie_h, bu, bp, bn, pos_o, neg_o, reg_o,
             biu, bip, binn, xuf, xue, xitp, xiep, xitn, xien, spos, sneg,
             regv, sem):
        w = lax.axis_index("s") * NC + lax.axis_index("c")
        per_w = BATCH // NW

        def g_body(g, racc):
            e0 = w * per_w + g * G
            di = (pltpu.async_copy(bu.at[pl.ds(e0, G)], biu, sem),
                  pltpu.async_copy(bp.at[pl.ds(e0, G)], bip, sem),
                  pltpu.async_copy(bn.at[pl.ds(e0 * N_NEG, G * N_NEG)], binn,
                                   sem))
            for d in di:
                d.wait()
            dg = (pltpu.async_copy(uf.at[biu], xuf, sem),
                  pltpu.async_copy(ue_h.at[biu], xue, sem),
                  pltpu.async_copy(itf.at[bip], xitp, sem),
                  pltpu.async_copy(ie_h.at[bip], xiep, sem),
                  pltpu.async_copy(itf.at[binn], xitn, sem),
                  pltpu.async_copy(ie_h.at[binn], xien, sem))
            for d in dg:
                d.wait()
            for e in range(G):
                pacc = jnp.zeros((L,), _f32)
                na = [jnp.zeros((L,), _f32) for _ in range(4)]
                ra = [jnp.zeros((L,), _f32) for _ in range(4)]
                for h0 in range(HS):
                    sl = pl.ds(h0 * L, L)
                    uv = xuf[e, sl]
                    pacc = pacc + uv * xitp[e, sl]
                    ev = xue[e, sl]
                    pv = xiep[e, sl]
                    ra[0] = ra[0] + ev * ev
                    ra[1] = ra[1] + pv * pv
                    for jn in range(N_NEG):
                        na[jn % 4] = na[jn % 4] + uv * xitn[e * N_NEG + jn, sl]
                        nv = xien[e * N_NEG + jn, sl]
                        ra[2 + jn % 2] = ra[2 + jn % 2] + nv * nv
                spos[e, pl.ds(0, L)] = pacc
                sneg[e, pl.ds(0, L)] = (na[0] + na[1]) + (na[2] + na[3])
                racc = racc + ((ra[0] + ra[1]) + (ra[2] + ra[3]))
            pltpu.sync_copy(spos, pos_o.at[pl.ds(e0, G)])
            pltpu.sync_copy(sneg, neg_o.at[pl.ds(e0, G)])
            return racc

        racc = lax.fori_loop(0, per_w // G, g_body, jnp.zeros((L,), _f32))
        regv[...] = racc
        pltpu.sync_copy(regv, reg_o.at[w])

    return pl.kernel(body, out_type=out_type, mesh=_mesh(),
                     compiler_params=_SC_PARAMS,
                     scratch_types=scratch, interpret=_INTERPRET)


def _tc_final(pos_ref, neg_ref, reg_ref, loss_ref, regl_ref):
    pos = jnp.sum(pos_ref[...], axis=1)
    neg = jnp.sum(neg_ref[...], axis=1) * (1.0 / N_NEG)
    d = neg - pos
    sp = jnp.maximum(d, 0.0) + jnp.log1p(jnp.exp(-jnp.abs(d)))
    loss_ref[...] = jnp.mean(sp).reshape(1, 1)
    regl_ref[...] = (0.5 * jnp.sum(reg_ref[...]) / BATCH).reshape(1, 1)


def _pad_edges(dst, src, val, nblk_in, spread_mod):
    et = nblk_in * FB
    n = dst.shape[0]
    padn = NW * et - n
    ar = jnp.arange(padn, dtype=_i32)
    sp = ar % spread_mod
    dst = jnp.concatenate([dst.astype(_i32), sp])
    src = jnp.concatenate([src.astype(_i32), sp])
    val = jnp.concatenate([val.astype(_f32), jnp.zeros((padn,), _f32)])
    return (dst.reshape(NW, nblk_in, FB), src.reshape(NW, nblk_in, FB),
            val.reshape(NW, nblk_in, FB))


def kernel(user_emb, item_emb, a_indices, a_values, s_indices, s_values,
           item_r, item_c, enhance_weight, item_degree,
           batch_user, batch_pos, batch_neg):
    x0 = jnp.concatenate(
        [user_emb.astype(_f32), item_emb.astype(_f32),
         jnp.zeros((NT_PAD - NT, H), _f32)], axis=0)
    nbi_a = -(-(1000000 // NW) // FB)
    nbi_s = -(-(500000 // NW) // FB)
    nbi_i = -(-(200000 // NW) // FB)

    ad, asx, av = _pad_edges(a_indices[0], a_indices[1], a_values, nbi_a,
                             CHUNK)
    bs_a, bd_a, bv_a, cn_a = _make_prepass(NB_A, nbi_a, 0)(ad, asx, av)
    spmm_a = _make_spmm(NB_A, "plain")
    y1 = spmm_a(x0, bs_a, bd_a, bv_a, cn_a)
    y2 = spmm_a(y1, bs_a, bd_a, bv_a, cn_a)

    sd, ssx, sv = _pad_edges(s_indices[0], s_indices[1], s_values, nbi_s,
                             CHUNK)
    bs_s, bd_s, bv_s, cn_s = _make_prepass(NB_S, nbi_s, 0)(sd, ssx, sv)
    uf = _make_spmm(NB_S, "users")(y2, bs_s, bd_s, bv_s, cn_s)

    idd, isx, iv = _pad_edges(item_r, item_c, enhance_weight, nbi_i, CHUNK)
    bs_i, bd_i, bv_i, cn_i = _make_prepass(NB_S, nbi_i, NU)(idd, isx, iv)
    degp = jnp.concatenate(
        [item_degree.astype(_f32), jnp.zeros((NU_PAD - NI,), _f32)])
    itf = _make_spmm(NB_S, "items")(y2, bs_i, bd_i, bv_i, cn_i, degp)

    pos_p, neg_p, reg_p = _make_bpr()(
        uf, itf, user_emb.astype(_f32), item_emb.astype(_f32),
        batch_user.astype(_i32), batch_pos.astype(_i32),
        batch_neg.reshape(-1).astype(_i32))
    outs = pl.pallas_call(
        _tc_final,
        out_shape=(jax.ShapeDtypeStruct((1, 1), _f32),
                   jax.ShapeDtypeStruct((1, 1), _f32)),
        interpret=_INTERPRET,
    )(pos_p, neg_p, reg_p)
    return (outs[0][0, 0], outs[1][0, 0])


# speedup vs baseline: 6.4570x; 1.0619x over previous
import jax
import jax.numpy as jnp
from jax import lax
from jax.experimental import pallas as pl
from jax.experimental.pallas import tpu as pltpu
from jax.experimental.pallas import tpu_sc as plsc

NU = 50000
NI = 50000
NT = 100000
H = 64
L = 16
HS = H // L
NC = 2
NS = 16
NW = NC * NS
CHUNK = 16896
RPT = CHUNK // NS
NB_A = 6
NB_S = 4
NT_PAD = NB_A * CHUNK
NU_PAD = NB_S * CHUNK
FB = 512
EBLK = 256
BATCH = 4096
N_NEG = 10
CONV = 10.0
_INTERPRET = False

_f32 = jnp.float32
_i32 = jnp.int32
_SC_PARAMS = pltpu.CompilerParams(needs_layout_passes=False,
                                  use_tc_tiling_on_sc=False)


def _mesh():
    return plsc.VectorSubcoreMesh(core_axis_name="c", subcore_axis_name="s",
                                  num_cores=NC, num_subcores=NS)


def _scal(x):
    x = jnp.asarray(x)
    return jnp.max(x) if x.ndim else x


def _lanes():
    return lax.broadcasted_iota(_i32, (L,), 0)


def _make_prepass(nb, nblk_in, src_off):
    cap = (nblk_in + 1) * FB
    out_type = (
        jax.ShapeDtypeStruct((nb, NW, cap), _i32),
        jax.ShapeDtypeStruct((nb, NW, cap), _i32),
        jax.ShapeDtypeStruct((nb, NW, cap), _f32),
        jax.ShapeDtypeStruct((NW, L), _i32),
    )
    scratch = [
        pltpu.VMEM((FB,), _i32),
        pltpu.VMEM((FB,), _i32),
        pltpu.VMEM((FB,), _f32),
    ]
    scratch += [pltpu.VMEM((2 * FB,), _i32) for _ in range(nb)]
    scratch += [pltpu.VMEM((2 * FB,), _i32) for _ in range(nb)]
    scratch += [pltpu.VMEM((2 * FB,), _f32) for _ in range(nb)]
    scratch.append(pltpu.VMEM((L,), _i32))
    scratch.append(pltpu.SemaphoreType.DMA)

    def body(dst_h, src_h, val_h, bsrc, bdst, bval, cnts, ind, ins, inv, *rest):
        stg_s = rest[0:nb]
        stg_d = rest[nb:2 * nb]
        stg_v = rest[2 * nb:3 * nb]
        cntv = rest[3 * nb]
        semp = rest[3 * nb + 1]
        w = lax.axis_index("s") * NC + lax.axis_index("c")
        pos = _lanes()

        def flush(b, nf):
            ds = (pltpu.async_copy(stg_s[b].at[pl.ds(0, FB)],
                                   bsrc.at[b, w, pl.ds(nf * FB, FB)], semp),
                  pltpu.async_copy(stg_d[b].at[pl.ds(0, FB)],
                                   bdst.at[b, w, pl.ds(nf * FB, FB)], semp),
                  pltpu.async_copy(stg_v[b].at[pl.ds(0, FB)],
                                   bval.at[b, w, pl.ds(nf * FB, FB)], semp))
            for d in ds:
                d.wait()

        def blk(j, carry):
            ptrs = list(carry[:nb])
            nfs = list(carry[nb:])
            ds = (pltpu.async_copy(dst_h.at[w, j], ind, semp),
                  pltpu.async_copy(src_h.at[w, j], ins, semp),
                  pltpu.async_copy(val_h.at[w, j], inv, semp))
            for d in ds:
                d.wait()
            for v in range(FB // L):
                sl = pl.ds(v * L, L)
                dv = ind[sl]
                sv = ins[sl] + src_off if src_off else ins[sl]
                vv = inv[sl]
                bv = lax.div(dv, jnp.int32(CHUNK))
                dl = dv - bv * CHUNK
                ms = [bv == b for b in range(nb)]
                incls = [plsc.cumsum(m.astype(_i32)) for m in ms]
                cnts_v = [plsc.all_reduce_population_count(m) for m in ms]
                for b in range(nb):
                    tgt = ptrs[b] + incls[b] - 1
                    plsc.store_scatter(stg_s[b], [tgt], sv, mask=ms[b])
                    plsc.store_scatter(stg_d[b], [tgt], dl, mask=ms[b])
                    plsc.store_scatter(stg_v[b], [tgt], vv, mask=ms[b])
                    ptrs[b] = ptrs[b] + cnts_v[b]
            for b in range(nb):
                ptr_s = jnp.max(ptrs[b])
                fl = ptr_s >= FB

                @pl.when(fl)
                def _(b=b, nf=nfs[b]):
                    flush(b, nf)
                    for v in range(FB // L):
                        lo = pl.ds(v * L, L)
                        hi = pl.ds(FB + v * L, L)
                        stg_s[b][lo] = stg_s[b][hi]
                        stg_d[b][lo] = stg_d[b][hi]
                        stg_v[b][lo] = stg_v[b][hi]

                ptrs[b] = jnp.where(fl, ptrs[b] - FB, ptrs[b])
                nfs[b] = jnp.where(fl, nfs[b] + 1, nfs[b])
            return tuple(ptrs) + tuple(nfs)

        init = tuple(jnp.zeros((L,), _i32) for _ in range(nb)) + tuple(
            jnp.int32(0) for _ in range(nb))
        carry = lax.fori_loop(0, nblk_in, blk, init)
        ptrs = [jnp.max(p) for p in carry[:nb]]
        nfs = carry[nb:]
        cv = jnp.zeros((L,), _i32)
        for b in range(nb):
            ptr = ptrs[b]
            for v in range(FB // L):
                sl = pl.ds(v * L, L)
                gpos = pos + v * L
                mi = (gpos >= ptr).astype(_i32)
                mf = mi.astype(_f32)
                stg_v[b][sl] = stg_v[b][sl] * (1.0 - mf)
                stg_d[b][sl] = stg_d[b][sl] * (1 - mi) + gpos * mi
                stg_s[b][sl] = stg_s[b][sl] * (1 - mi) + gpos * mi
            flush(b, nfs[b])
            cv = jnp.where(pos == b,
                           jnp.where(ptr > 0, nfs[b] + 1, nfs[b]), cv)
        cntv[...] = cv
        pltpu.sync_copy(cntv, cnts.at[w])

    return pl.kernel(body, out_type=out_type, mesh=_mesh(),
                     compiler_params=_SC_PARAMS,
                     scratch_types=scratch, interpret=_INTERPRET)


def _make_spmm(nb, mode):
    nbs = nb // NC
    NBUF = 2
    out_type = jax.ShapeDtypeStruct((nb * CHUNK, H), _f32)
    scratch = [pltpu.VMEM_SHARED((CHUNK, H), _f32)]
    scratch += [pltpu.VMEM((EBLK,), _i32) for _ in range(NBUF)]
    scratch += [pltpu.VMEM((EBLK,), _i32) for _ in range(NBUF)]
    scratch += [pltpu.VMEM((EBLK,), _f32) for _ in range(NBUF)]
    scratch += [pltpu.VMEM((EBLK, H), _f32) for _ in range(NBUF)]
    scratch += [pltpu.VMEM((32, H), _f32), pltpu.VMEM((L,), _i32)]
    scratch += [pltpu.SemaphoreType.DMA for _ in range(3 * NBUF)]
    if mode != "plain":
        scratch += [pltpu.VMEM((32, H), _f32), pltpu.VMEM((32, H), _f32)]
    if mode == "items":
        scratch.append(pltpu.VMEM((32,), _f32))

    def body(*args):
        x_h, bsrc, bdst, bval, cnts = args[:5]
        args = args[5:]
        if mode == "items":
            deg_h = args[0]
            args = args[1:]
        out = args[0]
        args = args[1:]
        accum = args[0]
        esrcs = args[1:1 + NBUF]
        edsts = args[1 + NBUF:1 + 2 * NBUF]
        evals = args[1 + 2 * NBUF:1 + 3 * NBUF]
        rowss = args[1 + 3 * NBUF:1 + 4 * NBUF]
        zbuf = args[1 + 4 * NBUF]
        cntv = args[2 + 4 * NBUF]
        base = 3 + 4 * NBUF
        sem_e = args[base:base + NBUF]
        sem_g = args[base + NBUF:base + 2 * NBUF]
        sem_s = args[base + 2 * NBUF:base + 3 * NBUF]
        rest = args[base + 3 * NBUF:]
        if mode != "plain":
            abuf, ybuf = rest[0], rest[1]
        if mode == "items":
            dbuf = rest[2]
        c = lax.axis_index("c")
        s = lax.axis_index("s")
        pos = _lanes()
        zero = jnp.zeros((L,), _f32)
        for r0 in range(32):
            for h0 in range(HS):
                zbuf[r0, pl.ds(h0 * L, L)] = zero
        for k in range(nbs):
            b = c * nbs + k

            def zr(t, _):
                pltpu.sync_copy(zbuf, accum.at[pl.ds(s * RPT + t * 32, 32)])
                return 0

            lax.fori_loop(0, RPT // 32, zr, 0)
            plsc.subcore_barrier()
            for rr in range(NW // NS):
                r = s * (NW // NS) + rr
                pltpu.sync_copy(cnts.at[r], cntv)
                nblk = jnp.max(jnp.where(pos == b, cntv[...], 0)) * (FB // EBLK)

                def edge_descs(j, p, b=b, r=r):
                    w = pl.ds(j * EBLK, EBLK)
                    return (
                        pltpu.make_async_copy(bsrc.at[b, r, w], esrcs[p],
                                              sem_e[p]),
                        pltpu.make_async_copy(bdst.at[b, r, w], edsts[p],
                                              sem_e[p]),
                        pltpu.make_async_copy(bval.at[b, r, w], evals[p],
                                              sem_e[p]),
                    )

                def start_edges(j, p):
                    for d in edge_descs(j, p):
                        d.start()

                def wait_edges(j, p):
                    for d in edge_descs(j, p):
                        d.wait()

                def gather_desc(p):
                    return pltpu.make_async_copy(x_h.at[esrcs[p]], rowss[p],
                                                 sem_g[p])

                def scatter_desc(p):
                    return pltpu.make_async_copy(rowss[p],
                                                 accum.at[edsts[p]], sem_s[p])

                @pl.when(nblk > 0)
                def _():
                    for jj in range(NBUF - 1):
                        start_edges(jj, jj)
                    wait_edges(0, 0)
                    gather_desc(0).start()

                def jgroup(j2, _):
                    for p in range(NBUF):
                        j = j2 * NBUF + p
                        q = (p + 1) % NBUF
                        gather_desc(p).wait()

                        @pl.when(j + NBUF - 1 < nblk)
                        def _(j=j, p=p):
                            start_edges(j + NBUF - 1, (p + NBUF - 1) % NBUF)

                        @pl.when(j + 1 < nblk)
                        def _(j=j, q=q):
                            wait_edges(j + 1, q)

                        cond = j + 1 < nblk
                        if p < NBUF - 1:
                            cond = jnp.logical_and(cond, j2 > 0)

                        @pl.when(cond)
                        def _(q=q):
                            scatter_desc(q).wait()

                        @pl.when(j + 1 < nblk)
                        def _(q=q):
                            gather_desc(q).start()

                        def scale(g, p=p):
                            for t in range(L):
                                row = g * L + t
                                wv = plsc.load_gather(
                                    evals[p], [jnp.full((L,), row, _i32)])
                                vals = [rowss[p][row, pl.ds(h0 * L, L)]
                                        for h0 in range(HS)]
                                for h0 in range(HS):
                                    rowss[p][row, pl.ds(h0 * L, L)] = (
                                        vals[h0] * wv)

                        plsc.parallel_loop(0, EBLK // L, unroll=2)(scale)
                        scatter_desc(p).start(add=True)
                    return 0

                lax.fori_loop(0, nblk // NBUF, jgroup, 0)

                @pl.when(nblk > 0)
                def _():
                    for q in range(1, NBUF):
                        scatter_desc(q).wait()

            plsc.subcore_barrier()
            if mode == "plain":
                pltpu.sync_copy(accum.at[pl.ds(s * RPT, RPT)],
                                out.at[pl.ds(b * CHUNK + s * RPT, RPT)])
            else:
                yoff = NU if mode == "items" else 0

                def ep(t, _, b=b):
                    lo = s * RPT + t * 32
                    glob = b * CHUNK + lo
                    pltpu.sync_copy(accum.at[pl.ds(lo, 32)], abuf)
                    pltpu.sync_copy(x_h.at[pl.ds(yoff + glob, 32)], ybuf)
                    if mode == "items":
                        pltpu.sync_copy(deg_h.at[pl.ds(glob, 32)], dbuf)
                    for row in range(32):
                        if mode == "users":
                            for h0 in range(HS):
                                sl = pl.ds(h0 * L, L)
                                abuf[row, sl] = (abuf[row, sl]
                                                 + ybuf[row, sl]) * 0.5
                        else:
                            dv = plsc.load_gather(
                                dbuf, [jnp.full((L,), row, _i32)])
                            sw = CONV / (CONV + jnp.exp(dv * (1.0 / CONV)))
                            for h0 in range(HS):
                                sl = pl.ds(h0 * L, L)
                                abuf[row, sl] = (ybuf[row, sl]
                                                 + sw * abuf[row, sl])
                    pltpu.sync_copy(abuf, out.at[pl.ds(glob, 32)])
                    return 0

                if mode == "items":
                    @pl.when(b * CHUNK < NI)
                    def _(b=b):
                        lax.fori_loop(0, RPT // 32, ep, 0)
                else:
                    lax.fori_loop(0, RPT // 32, ep, 0)
            plsc.subcore_barrier()

    return pl.kernel(body, out_type=out_type, mesh=_mesh(),
                     compiler_params=_SC_PARAMS,
                     scratch_types=scratch, interpret=_INTERPRET)


def _make_bpr():
    G = 16
    out_type = (
        jax.ShapeDtypeStruct((BATCH, L), _f32),
        jax.ShapeDtypeStruct((BATCH, L), _f32),
        jax.ShapeDtypeStruct((NW, L), _f32),
    )
    scratch = [
        pltpu.VMEM((G,), _i32),
        pltpu.VMEM((G,), _i32),
        pltpu.VMEM((G * N_NEG,), _i32),
        pltpu.VMEM((G, H), _f32),
        pltpu.VMEM((G, H), _f32),
        pltpu.VMEM((G, H), _f32),
        pltpu.VMEM((G, H), _f32),
        pltpu.VMEM((G * N_NEG, H), _f32),
        pltpu.VMEM((G * N_NEG, H), _f32),
        pltpu.VMEM((G, L), _f32),
        pltpu.VMEM((G, L), _f32),
        pltpu.VMEM((L,), _f32),
        pltpu.SemaphoreType.DMA,
    ]

    def body(uf, itf, ue_h, ie_h, bu, bp, bn, pos_o, neg_o, reg_o,
             biu, bip, binn, xuf, xue, xitp, xiep, xitn, xien, spos, sneg,
             regv, sem):
        w = lax.axis_index("s") * NC + lax.axis_index("c")
        per_w = BATCH // NW

        def g_body(g, racc):
            e0 = w * per_w + g * G
            di = (pltpu.async_copy(bu.at[pl.ds(e0, G)], biu, sem),
                  pltpu.async_copy(bp.at[pl.ds(e0, G)], bip, sem),
                  pltpu.async_copy(bn.at[pl.ds(e0 * N_NEG, G * N_NEG)], binn,
                                   sem))
            for d in di:
                d.wait()
            dg = (pltpu.async_copy(uf.at[biu], xuf, sem),
                  pltpu.async_copy(ue_h.at[biu], xue, sem),
                  pltpu.async_copy(itf.at[bip], xitp, sem),
                  pltpu.async_copy(ie_h.at[bip], xiep, sem),
                  pltpu.async_copy(itf.at[binn], xitn, sem),
                  pltpu.async_copy(ie_h.at[binn], xien, sem))
            for d in dg:
                d.wait()
            for e in range(G):
                pacc = jnp.zeros((L,), _f32)
                na = [jnp.zeros((L,), _f32) for _ in range(4)]
                ra = [jnp.zeros((L,), _f32) for _ in range(4)]
                for h0 in range(HS):
                    sl = pl.ds(h0 * L, L)
                    uv = xuf[e, sl]
                    pacc = pacc + uv * xitp[e, sl]
                    ev = xue[e, sl]
                    pv = xiep[e, sl]
                    ra[0] = ra[0] + ev * ev
                    ra[1] = ra[1] + pv * pv
                    for jn in range(N_NEG):
                        na[jn % 4] = na[jn % 4] + uv * xitn[e * N_NEG + jn, sl]
                        nv = xien[e * N_NEG + jn, sl]
                        ra[2 + jn % 2] = ra[2 + jn % 2] + nv * nv
                spos[e, pl.ds(0, L)] = pacc
                sneg[e, pl.ds(0, L)] = (na[0] + na[1]) + (na[2] + na[3])
                racc = racc + ((ra[0] + ra[1]) + (ra[2] + ra[3]))
            pltpu.sync_copy(spos, pos_o.at[pl.ds(e0, G)])
            pltpu.sync_copy(sneg, neg_o.at[pl.ds(e0, G)])
            return racc

        racc = lax.fori_loop(0, per_w // G, g_body, jnp.zeros((L,), _f32))
        regv[...] = racc
        pltpu.sync_copy(regv, reg_o.at[w])

    return pl.kernel(body, out_type=out_type, mesh=_mesh(),
                     compiler_params=_SC_PARAMS,
                     scratch_types=scratch, interpret=_INTERPRET)


def _tc_final(pos_ref, neg_ref, reg_ref, loss_ref, regl_ref):
    pos = jnp.sum(pos_ref[...], axis=1)
    neg = jnp.sum(neg_ref[...], axis=1) * (1.0 / N_NEG)
    d = neg - pos
    sp = jnp.maximum(d, 0.0) + jnp.log1p(jnp.exp(-jnp.abs(d)))
    loss_ref[...] = jnp.mean(sp).reshape(1, 1)
    regl_ref[...] = (0.5 * jnp.sum(reg_ref[...]) / BATCH).reshape(1, 1)


def _pad_edges(dst, src, val, nblk_in, spread_mod):
    et = nblk_in * FB
    n = dst.shape[0]
    padn = NW * et - n
    ar = jnp.arange(padn, dtype=_i32)
    sp = ar % spread_mod
    dst = jnp.concatenate([dst.astype(_i32), sp])
    src = jnp.concatenate([src.astype(_i32), sp])
    val = jnp.concatenate([val.astype(_f32), jnp.zeros((padn,), _f32)])
    return (dst.reshape(NW, nblk_in, FB), src.reshape(NW, nblk_in, FB),
            val.reshape(NW, nblk_in, FB))


def kernel(user_emb, item_emb, a_indices, a_values, s_indices, s_values,
           item_r, item_c, enhance_weight, item_degree,
           batch_user, batch_pos, batch_neg):
    x0 = jnp.concatenate(
        [user_emb.astype(_f32), item_emb.astype(_f32),
         jnp.zeros((NT_PAD - NT, H), _f32)], axis=0)
    nbi_a = -(-(1000000 // NW) // FB)
    nbi_s = -(-(500000 // NW) // FB)
    nbi_i = -(-(200000 // NW) // FB)

    ad, asx, av = _pad_edges(a_indices[0], a_indices[1], a_values, nbi_a,
                             CHUNK)
    bs_a, bd_a, bv_a, cn_a = _make_prepass(NB_A, nbi_a, 0)(ad, asx, av)
    spmm_a = _make_spmm(NB_A, "plain")
    y1 = spmm_a(x0, bs_a, bd_a, bv_a, cn_a)
    y2 = spmm_a(y1, bs_a, bd_a, bv_a, cn_a)

    sd, ssx, sv = _pad_edges(s_indices[0], s_indices[1], s_values, nbi_s,
                             CHUNK)
    bs_s, bd_s, bv_s, cn_s = _make_prepass(NB_S, nbi_s, 0)(sd, ssx, sv)
    uf = _make_spmm(NB_S, "users")(y2, bs_s, bd_s, bv_s, cn_s)

    idd, isx, iv = _pad_edges(item_r, item_c, enhance_weight, nbi_i, CHUNK)
    bs_i, bd_i, bv_i, cn_i = _make_prepass(NB_S, nbi_i, NU)(idd, isx, iv)
    degp = jnp.concatenate(
        [item_degree.astype(_f32), jnp.zeros((NU_PAD - NI,), _f32)])
    itf = _make_spmm(NB_S, "items")(y2, bs_i, bd_i, bv_i, cn_i, degp)

    pos_p, neg_p, reg_p = _make_bpr()(
        uf, itf, user_emb.astype(_f32), item_emb.astype(_f32),
        batch_user.astype(_i32), batch_pos.astype(_i32),
        batch_neg.reshape(-1).astype(_i32))
    outs = pl.pallas_call(
        _tc_final,
        out_shape=(jax.ShapeDtypeStruct((1, 1), _f32),
                   jax.ShapeDtypeStruct((1, 1), _f32)),
        interpret=_INTERPRET,
    )(pos_p, neg_p, reg_p)
    return (outs[0][0, 0], outs[1][0, 0])
